# baseline (speedup 1.0000x reference)
import functools

import jax
import jax.numpy as jnp
from jax import lax
from jax.experimental import pallas as pl
from jax.experimental.pallas import tpu as pltpu
from jax.experimental.pallas import tpu_sc as plsc

N = 16384
D = 16
DA = D + 2
K = 1024
CH = 2048
NSTEPS = N // CH
NC, NS = 2, 16
NW = NC * NS
B_PER_W = K // NW


def _argmin_body(x_ref, c_ref, val_ref, idx_ref):
    j = pl.program_id(0)
    xb = x_ref[...]
    cb = c_ref[...]
    xn2h = 0.5 * jnp.sum(xb * xb, axis=1, keepdims=True)
    hi = xn2h.astype(jnp.bfloat16).astype(jnp.float32)
    lo = xn2h - hi
    xa = jnp.concatenate([xb, hi, lo], axis=1)
    s = lax.dot_general(xa, cb, (((1,), (1,)), ((), ())),
                        preferred_element_type=jnp.float32,
                        precision=lax.Precision.HIGHEST)
    minv = jnp.min(s, axis=0, keepdims=True)
    rows = lax.broadcasted_iota(jnp.int32, (CH, K), 0)
    cand = jnp.where(s == minv, rows, jnp.int32(N))
    mini = jnp.min(cand, axis=0, keepdims=True) + j * CH

    @pl.when(j == 0)
    def _():
        val_ref[...] = minv
        idx_ref[...] = mini

    @pl.when(j > 0)
    def _():
        better = minv < val_ref[...]
        val_ref[...] = jnp.where(better, minv, val_ref[...])
        idx_ref[...] = jnp.where(better, mini, idx_ref[...])


_argmin_call = pl.pallas_call(
    _argmin_body,
    grid=(NSTEPS,),
    in_specs=[pl.BlockSpec((CH, D), lambda j: (j, 0)),
              pl.BlockSpec((K, DA), lambda j: (0, 0))],
    out_specs=[pl.BlockSpec((1, K), lambda j: (0, 0)),
               pl.BlockSpec((1, K), lambda j: (0, 0))],
    out_shape=[jax.ShapeDtypeStruct((1, K), jnp.float32),
               jax.ShapeDtypeStruct((1, K), jnp.int32)],
)


def _gather_body(table_hbm, idx_hbm, out_hbm, idx_v, rows_v, sem):
    wid = lax.axis_index("s") * NC + lax.axis_index("c")
    base = wid * B_PER_W
    pltpu.sync_copy(idx_hbm.at[pl.ds(base, B_PER_W)], idx_v)
    pltpu.async_copy(table_hbm.at[idx_v], rows_v, sem).wait()
    pltpu.sync_copy(rows_v, out_hbm.at[pl.ds(base, B_PER_W)])


@functools.cache
def _make_gather_call():
    return pl.kernel(
        _gather_body,
        out_type=jax.ShapeDtypeStruct((K, D), jnp.float32),
        mesh=plsc.VectorSubcoreMesh(core_axis_name="c", subcore_axis_name="s",
                                    num_cores=NC, num_subcores=NS),
        scratch_types=[
            pltpu.VMEM((B_PER_W,), jnp.int32),
            pltpu.VMEM((B_PER_W, D), jnp.float32),
            pltpu.SemaphoreType.DMA,
        ],
        compiler_params=pltpu.CompilerParams(use_tc_tiling_on_sc=False),
    )


def kernel(x, cluster_centers):
    x2d = x.reshape(N, D)
    caug = jnp.concatenate(
        [-cluster_centers, jnp.ones((K, 2), jnp.float32)], axis=1)
    _, idx = _argmin_call(x2d, caug)
    selected = _make_gather_call()(x2d, idx.reshape(K))
    return selected.reshape(1, K, D)


# speedup vs baseline: 1.2329x; 1.2329x over previous
import functools

import jax
import jax.numpy as jnp
from jax import lax
from jax.experimental import pallas as pl
from jax.experimental.pallas import tpu as pltpu
from jax.experimental.pallas import tpu_sc as plsc

N = 16384
D = 16
DA = 4 * D + 2
K = 1024
CH = 2048
NSTEPS = N // CH
NC, NS = 2, 16
NW = NC * NS
B_PER_W = K // NW


def _split_bf16(a):
    hi = a.astype(jnp.bfloat16)
    lo = (a - hi.astype(jnp.float32)).astype(jnp.bfloat16)
    return hi, lo


def _argmin_body(x_ref, c_ref, val_ref, idx_ref):
    j = pl.program_id(0)
    xb = x_ref[...]
    cb = c_ref[...]
    xn2h = 0.5 * jnp.sum(xb * xb, axis=1, keepdims=True)
    xhi, xlo = _split_bf16(xb)
    nhi, nlo = _split_bf16(xn2h)
    xa = jnp.concatenate([xhi, xlo, xhi, xlo, nhi, nlo], axis=1)
    s = lax.dot_general(xa, cb, (((1,), (1,)), ((), ())),
                        preferred_element_type=jnp.float32)
    minv = jnp.min(s, axis=0, keepdims=True)
    rows = lax.broadcasted_iota(jnp.int32, (CH, K), 0).astype(jnp.float32)
    cand = jnp.where(s == minv, rows, jnp.float32(N))
    mini = jnp.min(cand, axis=0, keepdims=True).astype(jnp.int32) + j * CH

    @pl.when(j == 0)
    def _():
        val_ref[...] = minv
        idx_ref[...] = mini

    @pl.when(j > 0)
    def _():
        better = minv < val_ref[...]
        val_ref[...] = jnp.where(better, minv, val_ref[...])
        idx_ref[...] = jnp.where(better, mini, idx_ref[...])


_argmin_call = pl.pallas_call(
    _argmin_body,
    grid=(NSTEPS,),
    in_specs=[pl.BlockSpec((CH, D), lambda j: (j, 0)),
              pl.BlockSpec((K, DA), lambda j: (0, 0))],
    out_specs=[pl.BlockSpec((1, K), lambda j: (0, 0)),
               pl.BlockSpec((1, K), lambda j: (0, 0))],
    out_shape=[jax.ShapeDtypeStruct((1, K), jnp.float32),
               jax.ShapeDtypeStruct((1, K), jnp.int32)],
)


def _augment_centers(cluster_centers):
    neg = -cluster_centers
    chi = neg.astype(jnp.bfloat16)
    clo = (neg - chi.astype(jnp.float32)).astype(jnp.bfloat16)
    ones = jnp.ones((K, 1), jnp.bfloat16)
    return jnp.concatenate([chi, chi, clo, clo, ones, ones], axis=1)


def _gather_body(table_hbm, idx_hbm, out_hbm, idx_v, rows_v, sem):
    wid = lax.axis_index("s") * NC + lax.axis_index("c")
    base = wid * B_PER_W
    pltpu.sync_copy(idx_hbm.at[pl.ds(base, B_PER_W)], idx_v)
    pltpu.async_copy(table_hbm.at[idx_v], rows_v, sem).wait()
    pltpu.sync_copy(rows_v, out_hbm.at[pl.ds(base, B_PER_W)])


@functools.cache
def _make_gather_call():
    return pl.kernel(
        _gather_body,
        out_type=jax.ShapeDtypeStruct((K, D), jnp.float32),
        mesh=plsc.VectorSubcoreMesh(core_axis_name="c", subcore_axis_name="s",
                                    num_cores=NC, num_subcores=NS),
        scratch_types=[
            pltpu.VMEM((B_PER_W,), jnp.int32),
            pltpu.VMEM((B_PER_W, D), jnp.float32),
            pltpu.SemaphoreType.DMA,
        ],
        compiler_params=pltpu.CompilerParams(use_tc_tiling_on_sc=False),
    )


def kernel(x, cluster_centers):
    x2d = x.reshape(N, D)
    caug = _augment_centers(cluster_centers)
    _, idx = _argmin_call(x2d, caug)
    selected = _make_gather_call()(x2d, idx.reshape(K))
    return selected.reshape(1, K, D)
